# Initial kernel scaffold; baseline (speedup 1.0000x reference)
import jax
import jax.numpy as jnp
from jax.experimental import pallas as pl


def kernel(latents_mean, embedding_weight):
    raise NotImplementedError("write your pallas kernel here")


# speedup vs baseline: 9.7223x; 9.7223x over previous
import functools

import jax
import jax.numpy as jnp
from jax import lax
from jax.experimental import pallas as pl
from jax.experimental.pallas import tpu as pltpu
from jax.experimental.pallas import tpu_sc as plsc

KK = 8192
DD = 256
NT = 8192
BETA = 0.25
DELTA = 1.0

BN = 1024
BK = 1024
NI = NT // BN
NJ = KK // BK


def _dist_argmin_body(zsq_ref, esq_ref, z_ref, e_ref, dist_ref, inds_ref,
                      minv_ref, mini_ref):
    j = pl.program_id(1)

    @pl.when(j == 0)
    def _():
        minv_ref[...] = jnp.full((BN, 1), jnp.inf, jnp.float32)
        mini_ref[...] = jnp.zeros((BN, 1), jnp.int32)

    z = z_ref[...]
    e = e_ref[pl.ds(j * BK, BK), :]
    mm = lax.dot_general(z, e, (((1,), (1,)), ((), ())),
                         preferred_element_type=jnp.float32)
    d = (zsq_ref[...] + esq_ref[:, pl.ds(j * BK, BK)]) - 2.0 * mm
    dist_ref[...] = d

    rmin = jnp.min(d, axis=1, keepdims=True)
    colid = lax.broadcasted_iota(jnp.int32, (BN, BK), 1) + j * BK
    rarg = jnp.min(jnp.where(d == rmin, colid, KK), axis=1, keepdims=True)

    better = rmin < minv_ref[...]
    mini_ref[...] = jnp.where(better, rarg, mini_ref[...])
    minv_ref[...] = jnp.where(better, rmin, minv_ref[...])

    @pl.when(j == NJ - 1)
    def _():
        inds_ref[...] = mini_ref[...]


_dist_argmin = pl.pallas_call(
    _dist_argmin_body,
    grid=(NI, NJ),
    in_specs=[
        pl.BlockSpec((BN, 1), lambda i, j: (i, 0)),
        pl.BlockSpec((1, KK), lambda i, j: (0, 0)),
        pl.BlockSpec((BN, DD), lambda i, j: (i, 0)),
        pl.BlockSpec((KK, DD), lambda i, j: (0, 0)),
    ],
    out_specs=[
        pl.BlockSpec((BN, BK), lambda i, j: (i, j)),
        pl.BlockSpec((BN, 1), lambda i, j: (i, 0)),
    ],
    out_shape=[
        jax.ShapeDtypeStruct((NT, KK), jnp.float32),
        jax.ShapeDtypeStruct((NT, 1), jnp.int32),
    ],
    scratch_shapes=[
        pltpu.VMEM((BN, 1), jnp.float32),
        pltpu.VMEM((BN, 1), jnp.int32),
    ],
)


_NW = 32
BPW = NT // _NW
NCH = BPW // 128


@functools.cache
def _build_sc_gather_hist():
    info = plsc.get_sparse_core_info()
    nc, ns = info.num_cores, info.num_subcores
    assert nc * ns == _NW
    mesh = plsc.VectorSubcoreMesh(core_axis_name="c", subcore_axis_name="s")

    @functools.partial(
        pl.kernel,
        mesh=mesh,
        out_type=[
            jax.ShapeDtypeStruct((_NW, BPW, DD), jnp.float32),
            jax.ShapeDtypeStruct((nc, KK), jnp.float32),
        ],
        scratch_types=[
            pltpu.VMEM((NCH, 128), jnp.int32),
            pltpu.VMEM((BPW, DD), jnp.float32),
            pltpu.VMEM((128,), jnp.float32),
            pltpu.VMEM((KK,), jnp.float32),
            pltpu.VMEM_SHARED((KK,), jnp.float32),
            pltpu.SemaphoreType.DMA,
        ],
    )
    def _sc_gather_hist(emb_hbm, idx_hbm, out_hbm, cnt_hbm,
                        idx_v, rows_v, ones_v, zbuf_v, hist_sh, sem):
        c = lax.axis_index("c")
        s = lax.axis_index("s")
        wid = c * ns + s

        pltpu.sync_copy(idx_hbm.at[wid], idx_v)

        def _ones_body(t, carry):
            ones_v[pl.ds(t * 16, 16)] = jnp.full((16,), 1.0, jnp.float32)
            return carry
        lax.fori_loop(0, 128 // 16, _ones_body, 0)

        @pl.when(s == 0)
        def _():
            def _z_body(t, carry):
                zbuf_v[pl.ds(t * 16, 16)] = jnp.zeros((16,), jnp.float32)
                return carry
            lax.fori_loop(0, KK // 16, _z_body, 0)
            pltpu.sync_copy(zbuf_v, hist_sh)

        copies = []
        for ch in range(NCH):
            copies.append(pltpu.async_copy(
                emb_hbm.at[idx_v.at[ch]], rows_v.at[pl.ds(ch * 128, 128)], sem))
        for cp in copies:
            cp.wait()
        pltpu.sync_copy(rows_v, out_hbm.at[wid])

        plsc.subcore_barrier()
        for ch in range(NCH):
            pltpu.sync_copy(ones_v, hist_sh.at[idx_v.at[ch]], add=True)
        plsc.subcore_barrier()

        @pl.when(s == 0)
        def _():
            pltpu.sync_copy(hist_sh, cnt_hbm.at[c])

    return _sc_gather_hist


def _final_body(z_ref, q_ref, cnt_ref, qst_ref, loss_ref, perp_ref, acc_ref):
    b = pl.program_id(0)
    nb = pl.num_programs(0)
    z = z_ref[...]
    q = q_ref[...]
    dqz = q - z
    qst_ref[...] = z + dqz

    @pl.when(b == 0)
    def _():
        acc_ref[0] = 0.0
        cnt = cnt_ref[0:1, :] + cnt_ref[1:2, :]
        avg = cnt * (1.0 / NT)
        ent = jnp.sum(avg * jnp.log(avg + 1e-10))
        perp_ref[0, 0] = jnp.exp(-ent)

    acc_ref[0] = acc_ref[0] + jnp.sum(dqz * dqz)

    @pl.when(b == nb - 1)
    def _():
        m = acc_ref[0] / (NT * DD)
        loss_ref[0, 0] = BETA * m + DELTA * m


_final = pl.pallas_call(
    _final_body,
    grid=(NI,),
    in_specs=[
        pl.BlockSpec((BN, DD), lambda i: (i, 0)),
        pl.BlockSpec((BN, DD), lambda i: (i, 0)),
        pl.BlockSpec((2, KK), lambda i: (0, 0)),
    ],
    out_specs=[
        pl.BlockSpec((BN, DD), lambda i: (i, 0)),
        pl.BlockSpec(memory_space=pltpu.SMEM),
        pl.BlockSpec(memory_space=pltpu.SMEM),
    ],
    out_shape=[
        jax.ShapeDtypeStruct((NT, DD), jnp.float32),
        jax.ShapeDtypeStruct((1, 1), jnp.float32),
        jax.ShapeDtypeStruct((1, 1), jnp.float32),
    ],
    scratch_shapes=[pltpu.SMEM((1,), jnp.float32)],
)


def kernel(latents_mean, embedding_weight):
    latents_shape = latents_mean.shape
    flat = latents_mean.reshape(NT, DD)
    zsq = jnp.sum(flat ** 2, axis=1, keepdims=True)
    esq = jnp.sum(embedding_weight ** 2, axis=1)[None, :]

    dist, inds = _dist_argmin(zsq, esq, flat, embedding_weight)

    idx3 = inds.reshape(_NW, NCH, 128)
    rows, counts = _build_sc_gather_hist()(embedding_weight, idx3)
    quantized = rows.reshape(NT, DD)

    qst, loss, perp = _final(flat, quantized, counts)

    return (qst.reshape(latents_shape), loss.reshape(()), perp.reshape(()),
            inds, dist)


# speedup vs baseline: 11.2049x; 1.1525x over previous
import functools

import jax
import jax.numpy as jnp
from jax import lax
from jax.experimental import pallas as pl
from jax.experimental.pallas import tpu as pltpu
from jax.experimental.pallas import tpu_sc as plsc

KK = 8192
DD = 256
NT = 8192
BETA = 0.25
DELTA = 1.0

BN = 2048
BK = 2048
NI = NT // BN
NJ = KK // BK


def _dist_argmin_body(zsq_ref, esq_ref, z2_ref, e_ref, dist_ref, inds_ref,
                      minv_ref, mini_ref):
    j = pl.program_id(1)

    @pl.when(j == 0)
    def _():
        minv_ref[...] = jnp.full((BN, 1), jnp.inf, jnp.float32)
        mini_ref[...] = jnp.zeros((BN, 1), jnp.int32)

    z2 = z2_ref[...]
    e = e_ref[pl.ds(j * BK, BK), :]
    mm2 = lax.dot_general(z2, e, (((1,), (1,)), ((), ())),
                          preferred_element_type=jnp.float32)
    d = (zsq_ref[...] + esq_ref[:, pl.ds(j * BK, BK)]) - mm2
    dist_ref[...] = d

    rmin = jnp.min(d, axis=1, keepdims=True)
    colid = lax.broadcasted_iota(jnp.int32, (BN, BK), 1)
    loc = jnp.min(jnp.where(d == rmin, colid, BK), axis=1, keepdims=True)
    rarg = loc + j * BK

    better = rmin < minv_ref[...]
    mini_ref[...] = jnp.where(better, rarg, mini_ref[...])
    minv_ref[...] = jnp.where(better, rmin, minv_ref[...])

    @pl.when(j == NJ - 1)
    def _():
        inds_ref[...] = mini_ref[...]


_dist_argmin = pl.pallas_call(
    _dist_argmin_body,
    grid=(NI, NJ),
    in_specs=[
        pl.BlockSpec((BN, 1), lambda i, j: (i, 0)),
        pl.BlockSpec((1, KK), lambda i, j: (0, 0)),
        pl.BlockSpec((BN, DD), lambda i, j: (i, 0)),
        pl.BlockSpec((KK, DD), lambda i, j: (0, 0)),
    ],
    out_specs=[
        pl.BlockSpec((BN, BK), lambda i, j: (i, j)),
        pl.BlockSpec((BN, 1), lambda i, j: (i, 0)),
    ],
    out_shape=[
        jax.ShapeDtypeStruct((NT, KK), jnp.float32),
        jax.ShapeDtypeStruct((NT, 1), jnp.int32),
    ],
    scratch_shapes=[
        pltpu.VMEM((BN, 1), jnp.float32),
        pltpu.VMEM((BN, 1), jnp.int32),
    ],
)


_NW = 32
BPW = NT // _NW
NCH = BPW // 128


@functools.cache
def _build_sc_gather_hist():
    info = plsc.get_sparse_core_info()
    nc, ns = info.num_cores, info.num_subcores
    assert nc * ns == _NW
    mesh = plsc.VectorSubcoreMesh(core_axis_name="c", subcore_axis_name="s")

    @functools.partial(
        pl.kernel,
        mesh=mesh,
        out_type=[
            jax.ShapeDtypeStruct((_NW, BPW, DD), jnp.float32),
            jax.ShapeDtypeStruct((nc, KK), jnp.float32),
        ],
        scratch_types=[
            pltpu.VMEM((NCH, 128), jnp.int32),
            pltpu.VMEM((BPW, DD), jnp.float32),
            pltpu.VMEM((128,), jnp.float32),
            pltpu.VMEM((KK,), jnp.float32),
            pltpu.VMEM_SHARED((KK,), jnp.float32),
            pltpu.SemaphoreType.DMA,
        ],
    )
    def _sc_gather_hist(emb_hbm, idx_hbm, out_hbm, cnt_hbm,
                        idx_v, rows_v, ones_v, zbuf_v, hist_sh, sem):
        c = lax.axis_index("c")
        s = lax.axis_index("s")
        wid = c * ns + s

        pltpu.sync_copy(idx_hbm.at[wid], idx_v)

        def _ones_body(t, carry):
            ones_v[pl.ds(t * 16, 16)] = jnp.full((16,), 1.0, jnp.float32)
            return carry
        lax.fori_loop(0, 128 // 16, _ones_body, 0)

        @pl.when(s == 0)
        def _():
            def _z_body(t, carry):
                zbuf_v[pl.ds(t * 16, 16)] = jnp.zeros((16,), jnp.float32)
                return carry
            lax.fori_loop(0, KK // 16, _z_body, 0)
            pltpu.sync_copy(zbuf_v, hist_sh)

        copies = []
        for ch in range(NCH):
            copies.append(pltpu.async_copy(
                emb_hbm.at[idx_v.at[ch]], rows_v.at[pl.ds(ch * 128, 128)], sem))
        for cp in copies:
            cp.wait()
        pltpu.sync_copy(rows_v, out_hbm.at[wid])

        plsc.subcore_barrier()
        for ch in range(NCH):
            pltpu.sync_copy(ones_v, hist_sh.at[idx_v.at[ch]], add=True)
        plsc.subcore_barrier()

        @pl.when(s == 0)
        def _():
            pltpu.sync_copy(hist_sh, cnt_hbm.at[c])

    return _sc_gather_hist


def _final_body(z_ref, q_ref, cnt_ref, qst_ref, loss_ref, perp_ref, acc_ref):
    b = pl.program_id(0)
    nb = pl.num_programs(0)
    z = z_ref[...]
    q = q_ref[...]
    dqz = q - z
    qst_ref[...] = z + dqz

    @pl.when(b == 0)
    def _():
        acc_ref[0] = 0.0
        cnt = cnt_ref[0:1, :] + cnt_ref[1:2, :]
        avg = cnt * (1.0 / NT)
        ent = jnp.sum(avg * jnp.log(avg + 1e-10))
        perp_ref[0, 0] = jnp.exp(-ent)

    acc_ref[0] = acc_ref[0] + jnp.sum(dqz * dqz)

    @pl.when(b == nb - 1)
    def _():
        m = acc_ref[0] / (NT * DD)
        loss_ref[0, 0] = BETA * m + DELTA * m


_final = pl.pallas_call(
    _final_body,
    grid=(NI,),
    in_specs=[
        pl.BlockSpec((BN, DD), lambda i: (i, 0)),
        pl.BlockSpec((BN, DD), lambda i: (i, 0)),
        pl.BlockSpec((2, KK), lambda i: (0, 0)),
    ],
    out_specs=[
        pl.BlockSpec((BN, DD), lambda i: (i, 0)),
        pl.BlockSpec(memory_space=pltpu.SMEM),
        pl.BlockSpec(memory_space=pltpu.SMEM),
    ],
    out_shape=[
        jax.ShapeDtypeStruct((NT, DD), jnp.float32),
        jax.ShapeDtypeStruct((1, 1), jnp.float32),
        jax.ShapeDtypeStruct((1, 1), jnp.float32),
    ],
    scratch_shapes=[pltpu.SMEM((1,), jnp.float32)],
)


def kernel(latents_mean, embedding_weight):
    latents_shape = latents_mean.shape
    flat = latents_mean.reshape(NT, DD)
    zsq = jnp.sum(flat ** 2, axis=1, keepdims=True)
    esq = jnp.sum(embedding_weight ** 2, axis=1)[None, :]

    dist, inds = _dist_argmin(zsq, esq, 2.0 * flat, embedding_weight)

    idx3 = inds.reshape(_NW, NCH, 128)
    rows, counts = _build_sc_gather_hist()(embedding_weight, idx3)
    quantized = rows.reshape(NT, DD)

    qst, loss, perp = _final(flat, quantized, counts)

    return (qst.reshape(latents_shape), loss.reshape(()), perp.reshape(()),
            inds, dist)


# speedup vs baseline: 11.8715x; 1.0595x over previous
import functools

import jax
import jax.numpy as jnp
from jax import lax
from jax.experimental import pallas as pl
from jax.experimental.pallas import tpu as pltpu
from jax.experimental.pallas import tpu_sc as plsc

KK = 8192
DD = 256
NT = 8192
BETA = 0.25
DELTA = 1.0

BN = 2048
BK = 2048
NI = NT // BN
NJ = KK // BK


def _dist_argmin_body(zsq_ref, esq_ref, z2_ref, e_ref, dist_ref, inds_ref,
                      minv_ref, mini_ref):
    j = pl.program_id(1)

    @pl.when(j == 0)
    def _():
        minv_ref[...] = jnp.full((BN, 1), jnp.inf, jnp.float32)
        mini_ref[...] = jnp.zeros((BN, 1), jnp.int32)

    z = z2_ref[...]
    z2 = z + z
    e = e_ref[pl.ds(j * BK, BK), :]
    mm2 = lax.dot_general(z2, e, (((1,), (1,)), ((), ())),
                          preferred_element_type=jnp.float32)
    d = (zsq_ref[...] + esq_ref[:, pl.ds(j * BK, BK)]) - mm2
    dist_ref[...] = d

    rmin = jnp.min(d, axis=1, keepdims=True)
    colid = lax.broadcasted_iota(jnp.int32, (BN, BK), 1)
    loc = jnp.min(jnp.where(d == rmin, colid, BK), axis=1, keepdims=True)
    rarg = loc + j * BK

    better = rmin < minv_ref[...]
    mini_ref[...] = jnp.where(better, rarg, mini_ref[...])
    minv_ref[...] = jnp.where(better, rmin, minv_ref[...])

    @pl.when(j == NJ - 1)
    def _():
        inds_ref[...] = mini_ref[...]


_dist_argmin = pl.pallas_call(
    _dist_argmin_body,
    grid=(NI, NJ),
    in_specs=[
        pl.BlockSpec((BN, 1), lambda i, j: (i, 0)),
        pl.BlockSpec((1, KK), lambda i, j: (0, 0)),
        pl.BlockSpec((BN, DD), lambda i, j: (i, 0)),
        pl.BlockSpec((KK, DD), lambda i, j: (0, 0)),
    ],
    out_specs=[
        pl.BlockSpec((BN, BK), lambda i, j: (i, j)),
        pl.BlockSpec((BN, 1), lambda i, j: (i, 0)),
    ],
    out_shape=[
        jax.ShapeDtypeStruct((NT, KK), jnp.float32),
        jax.ShapeDtypeStruct((NT, 1), jnp.int32),
    ],
    scratch_shapes=[
        pltpu.VMEM((BN, 1), jnp.float32),
        pltpu.VMEM((BN, 1), jnp.int32),
    ],
)


_NW = 32
BPW = NT // _NW
NCH = BPW // 128


@functools.cache
def _build_sc_gather_hist():
    info = plsc.get_sparse_core_info()
    nc, ns = info.num_cores, info.num_subcores
    assert nc * ns == _NW
    mesh = plsc.VectorSubcoreMesh(core_axis_name="c", subcore_axis_name="s")

    kps = KK // ns

    @functools.partial(
        pl.kernel,
        mesh=mesh,
        out_type=[
            jax.ShapeDtypeStruct((NT, DD), jnp.float32),
            jax.ShapeDtypeStruct((nc, KK), jnp.float32),
        ],
        scratch_types=[
            pltpu.VMEM((NCH, 128), jnp.int32),
            pltpu.VMEM((BPW, DD), jnp.float32),
            pltpu.VMEM((128,), jnp.float32),
            pltpu.VMEM((kps,), jnp.float32),
            pltpu.VMEM_SHARED((KK,), jnp.float32),
            pltpu.SemaphoreType.DMA,
        ],
    )
    def _sc_gather_hist(emb_hbm, idx_hbm, out_hbm, cnt_hbm,
                        idx_v, rows_v, ones_v, zbuf_v, hist_sh, sem):
        c = lax.axis_index("c")
        s = lax.axis_index("s")
        wid = c * ns + s

        pltpu.sync_copy(idx_hbm.at[wid], idx_v)

        def _ones_body(t, carry):
            ones_v[pl.ds(t * 16, 16)] = jnp.full((16,), 1.0, jnp.float32)
            return carry
        lax.fori_loop(0, 128 // 16, _ones_body, 0)

        def _z_body(t, carry):
            zbuf_v[pl.ds(t * 16, 16)] = jnp.zeros((16,), jnp.float32)
            return carry
        lax.fori_loop(0, kps // 16, _z_body, 0)
        pltpu.sync_copy(zbuf_v, hist_sh.at[pl.ds(s * kps, kps)])

        copies = []
        for ch in range(NCH):
            copies.append(pltpu.async_copy(
                emb_hbm.at[idx_v.at[ch]], rows_v.at[pl.ds(ch * 128, 128)], sem))
        for cp in copies:
            cp.wait()
        pltpu.sync_copy(rows_v, out_hbm.at[pl.ds(wid * BPW, BPW)])

        plsc.subcore_barrier()
        for ch in range(NCH):
            pltpu.sync_copy(ones_v, hist_sh.at[idx_v.at[ch]], add=True)
        plsc.subcore_barrier()

        @pl.when(s == 0)
        def _():
            pltpu.sync_copy(hist_sh, cnt_hbm.at[c])

    return _sc_gather_hist


def _final_body(z_ref, q_ref, cnt_ref, qst_ref, loss_ref, perp_ref, acc_ref):
    b = pl.program_id(0)
    nb = pl.num_programs(0)
    z = z_ref[...]
    q = q_ref[...]
    dqz = q - z
    qst_ref[...] = z + dqz

    @pl.when(b == 0)
    def _():
        acc_ref[0] = 0.0
        cnt = cnt_ref[0:1, :] + cnt_ref[1:2, :]
        avg = cnt * (1.0 / NT)
        ent = jnp.sum(avg * jnp.log(avg + 1e-10))
        perp_ref[0, 0] = jnp.exp(-ent)

    acc_ref[0] = acc_ref[0] + jnp.sum(dqz * dqz)

    @pl.when(b == nb - 1)
    def _():
        m = acc_ref[0] / (NT * DD)
        loss_ref[0, 0] = BETA * m + DELTA * m


_final = pl.pallas_call(
    _final_body,
    grid=(NI,),
    in_specs=[
        pl.BlockSpec((BN, DD), lambda i: (i, 0)),
        pl.BlockSpec((BN, DD), lambda i: (i, 0)),
        pl.BlockSpec((2, KK), lambda i: (0, 0)),
    ],
    out_specs=[
        pl.BlockSpec((BN, DD), lambda i: (i, 0)),
        pl.BlockSpec(memory_space=pltpu.SMEM),
        pl.BlockSpec(memory_space=pltpu.SMEM),
    ],
    out_shape=[
        jax.ShapeDtypeStruct((NT, DD), jnp.float32),
        jax.ShapeDtypeStruct((1, 1), jnp.float32),
        jax.ShapeDtypeStruct((1, 1), jnp.float32),
    ],
    scratch_shapes=[pltpu.SMEM((1,), jnp.float32)],
)


def kernel(latents_mean, embedding_weight):
    latents_shape = latents_mean.shape
    flat = latents_mean.reshape(NT, DD)
    zsq = jnp.sum(flat ** 2, axis=1, keepdims=True)
    esq = jnp.sum(embedding_weight ** 2, axis=1)[None, :]

    dist, inds = _dist_argmin(zsq, esq, flat, embedding_weight)

    idx3 = inds.reshape(_NW, NCH, 128)
    quantized, counts = _build_sc_gather_hist()(embedding_weight, idx3)

    qst, loss, perp = _final(flat, quantized, counts)

    return (qst.reshape(latents_shape), loss.reshape(()), perp.reshape(()),
            inds, dist)


# speedup vs baseline: 12.3308x; 1.0387x over previous
import functools

import jax
import jax.numpy as jnp
from jax import lax
from jax.experimental import pallas as pl
from jax.experimental.pallas import tpu as pltpu
from jax.experimental.pallas import tpu_sc as plsc

KK = 8192
DD = 256
NT = 8192
BETA = 0.25
DELTA = 1.0

BN = 512
BK = 8192
NI = NT // BN
NJ = KK // BK


def _dist_argmin_body(zsq_ref, esq_ref, z2_ref, e_ref, dist_ref, inds_ref,
                      minv_ref, mini_ref):
    j = pl.program_id(1)

    @pl.when(j == 0)
    def _():
        minv_ref[...] = jnp.full((BN, 1), jnp.inf, jnp.float32)
        mini_ref[...] = jnp.zeros((BN, 1), jnp.int32)

    z = z2_ref[...]
    z2 = z + z
    e = e_ref[pl.ds(j * BK, BK), :]
    mm2 = lax.dot_general(z2, e, (((1,), (1,)), ((), ())),
                          preferred_element_type=jnp.float32)
    d = (zsq_ref[...] + esq_ref[:, pl.ds(j * BK, BK)]) - mm2
    dist_ref[...] = d

    rmin = jnp.min(d, axis=1, keepdims=True)
    colid = lax.broadcasted_iota(jnp.int32, (BN, BK), 1)
    loc = jnp.min(jnp.where(d == rmin, colid, BK), axis=1, keepdims=True)
    rarg = loc + j * BK

    better = rmin < minv_ref[...]
    mini_ref[...] = jnp.where(better, rarg, mini_ref[...])
    minv_ref[...] = jnp.where(better, rmin, minv_ref[...])

    @pl.when(j == NJ - 1)
    def _():
        inds_ref[...] = mini_ref[...]


_dist_argmin = pl.pallas_call(
    _dist_argmin_body,
    grid=(NI, NJ),
    in_specs=[
        pl.BlockSpec((BN, 1), lambda i, j: (i, 0)),
        pl.BlockSpec((1, KK), lambda i, j: (0, 0)),
        pl.BlockSpec((BN, DD), lambda i, j: (i, 0)),
        pl.BlockSpec((KK, DD), lambda i, j: (0, 0)),
    ],
    out_specs=[
        pl.BlockSpec((BN, BK), lambda i, j: (i, j)),
        pl.BlockSpec((BN, 1), lambda i, j: (i, 0)),
    ],
    out_shape=[
        jax.ShapeDtypeStruct((NT, KK), jnp.float32),
        jax.ShapeDtypeStruct((NT, 1), jnp.int32),
    ],
    scratch_shapes=[
        pltpu.VMEM((BN, 1), jnp.float32),
        pltpu.VMEM((BN, 1), jnp.int32),
    ],
)


_NW = 32
BPW = NT // _NW
NCH = BPW // 128


@functools.cache
def _build_sc_gather_hist():
    info = plsc.get_sparse_core_info()
    nc, ns = info.num_cores, info.num_subcores
    assert nc * ns == _NW
    mesh = plsc.VectorSubcoreMesh(core_axis_name="c", subcore_axis_name="s")

    kps = KK // ns

    @functools.partial(
        pl.kernel,
        mesh=mesh,
        out_type=[
            jax.ShapeDtypeStruct((NT, DD), jnp.float32),
            jax.ShapeDtypeStruct((nc, KK), jnp.float32),
        ],
        scratch_types=[
            pltpu.VMEM((NCH, 128), jnp.int32),
            pltpu.VMEM((BPW, DD), jnp.float32),
            pltpu.VMEM((128,), jnp.float32),
            pltpu.VMEM((kps,), jnp.float32),
            pltpu.VMEM_SHARED((KK,), jnp.float32),
            pltpu.SemaphoreType.DMA,
        ],
    )
    def _sc_gather_hist(emb_hbm, idx_hbm, out_hbm, cnt_hbm,
                        idx_v, rows_v, ones_v, zbuf_v, hist_sh, sem):
        c = lax.axis_index("c")
        s = lax.axis_index("s")
        wid = c * ns + s

        pltpu.sync_copy(idx_hbm.at[wid], idx_v)

        def _ones_body(t, carry):
            ones_v[pl.ds(t * 16, 16)] = jnp.full((16,), 1.0, jnp.float32)
            return carry
        lax.fori_loop(0, 128 // 16, _ones_body, 0)

        def _z_body(t, carry):
            zbuf_v[pl.ds(t * 16, 16)] = jnp.zeros((16,), jnp.float32)
            return carry
        lax.fori_loop(0, kps // 16, _z_body, 0)
        pltpu.sync_copy(zbuf_v, hist_sh.at[pl.ds(s * kps, kps)])

        copies = []
        for ch in range(NCH):
            copies.append(pltpu.async_copy(
                emb_hbm.at[idx_v.at[ch]], rows_v.at[pl.ds(ch * 128, 128)], sem))
        for cp in copies:
            cp.wait()
        pltpu.sync_copy(rows_v, out_hbm.at[pl.ds(wid * BPW, BPW)])

        plsc.subcore_barrier()
        for ch in range(NCH):
            pltpu.sync_copy(ones_v, hist_sh.at[idx_v.at[ch]], add=True)
        plsc.subcore_barrier()

        @pl.when(s == 0)
        def _():
            pltpu.sync_copy(hist_sh, cnt_hbm.at[c])

    return _sc_gather_hist


def _final_body(z_ref, q_ref, cnt_ref, qst_ref, loss_ref, perp_ref, acc_ref):
    b = pl.program_id(0)
    nb = pl.num_programs(0)
    z = z_ref[...]
    q = q_ref[...]
    dqz = q - z
    qst_ref[...] = z + dqz

    @pl.when(b == 0)
    def _():
        acc_ref[0] = 0.0
        cnt = cnt_ref[0:1, :] + cnt_ref[1:2, :]
        avg = cnt * (1.0 / NT)
        ent = jnp.sum(avg * jnp.log(avg + 1e-10))
        perp_ref[0, 0] = jnp.exp(-ent)

    acc_ref[0] = acc_ref[0] + jnp.sum(dqz * dqz)

    @pl.when(b == nb - 1)
    def _():
        m = acc_ref[0] / (NT * DD)
        loss_ref[0, 0] = BETA * m + DELTA * m


_final = pl.pallas_call(
    _final_body,
    grid=(NI,),
    in_specs=[
        pl.BlockSpec((BN, DD), lambda i: (i, 0)),
        pl.BlockSpec((BN, DD), lambda i: (i, 0)),
        pl.BlockSpec((2, KK), lambda i: (0, 0)),
    ],
    out_specs=[
        pl.BlockSpec((BN, DD), lambda i: (i, 0)),
        pl.BlockSpec(memory_space=pltpu.SMEM),
        pl.BlockSpec(memory_space=pltpu.SMEM),
    ],
    out_shape=[
        jax.ShapeDtypeStruct((NT, DD), jnp.float32),
        jax.ShapeDtypeStruct((1, 1), jnp.float32),
        jax.ShapeDtypeStruct((1, 1), jnp.float32),
    ],
    scratch_shapes=[pltpu.SMEM((1,), jnp.float32)],
)


def kernel(latents_mean, embedding_weight):
    latents_shape = latents_mean.shape
    flat = latents_mean.reshape(NT, DD)
    zsq = jnp.sum(flat ** 2, axis=1, keepdims=True)
    esq = jnp.sum(embedding_weight ** 2, axis=1)[None, :]

    dist, inds = _dist_argmin(zsq, esq, flat, embedding_weight)

    idx3 = inds.reshape(_NW, NCH, 128)
    quantized, counts = _build_sc_gather_hist()(embedding_weight, idx3)

    qst, loss, perp = _final(flat, quantized, counts)

    return (qst.reshape(latents_shape), loss.reshape(()), perp.reshape(()),
            inds, dist)


# speedup vs baseline: 13.3910x; 1.0860x over previous
import functools

import jax
import jax.numpy as jnp
from jax import lax
from jax.experimental import pallas as pl
from jax.experimental.pallas import tpu as pltpu
from jax.experimental.pallas import tpu_sc as plsc

KK = 8192
DD = 256
NT = 8192
BETA = 0.25
DELTA = 1.0

BN = 512
BK = 8192
NI = NT // BN
NJ = KK // BK


def _dist_argmin_body(zsq_ref, esq_ref, z2_ref, e_ref, dist_ref, inds_ref,
                      mino_ref, minv_ref, mini_ref):
    j = pl.program_id(1)

    @pl.when(j == 0)
    def _():
        minv_ref[...] = jnp.full((BN, 1), jnp.inf, jnp.float32)
        mini_ref[...] = jnp.zeros((BN, 1), jnp.int32)

    z = z2_ref[...]
    z2 = z + z
    e = e_ref[pl.ds(j * BK, BK), :]
    mm2 = lax.dot_general(z2, e, (((1,), (1,)), ((), ())),
                          preferred_element_type=jnp.float32)
    d = (zsq_ref[...] + esq_ref[:, pl.ds(j * BK, BK)]) - mm2
    dist_ref[...] = d

    rmin = jnp.min(d, axis=1, keepdims=True)
    colid = lax.broadcasted_iota(jnp.int32, (BN, BK), 1)
    loc = jnp.min(jnp.where(d == rmin, colid, BK), axis=1, keepdims=True)
    rarg = loc + j * BK

    better = rmin < minv_ref[...]
    mini_ref[...] = jnp.where(better, rarg, mini_ref[...])
    minv_ref[...] = jnp.where(better, rmin, minv_ref[...])

    @pl.when(j == NJ - 1)
    def _():
        inds_ref[...] = mini_ref[...]
        mino_ref[...] = minv_ref[...]


_dist_argmin = pl.pallas_call(
    _dist_argmin_body,
    grid=(NI, NJ),
    in_specs=[
        pl.BlockSpec((BN, 1), lambda i, j: (i, 0)),
        pl.BlockSpec((1, KK), lambda i, j: (0, 0)),
        pl.BlockSpec((BN, DD), lambda i, j: (i, 0)),
        pl.BlockSpec((KK, DD), lambda i, j: (0, 0)),
    ],
    out_specs=[
        pl.BlockSpec((BN, BK), lambda i, j: (i, j)),
        pl.BlockSpec((BN, 1), lambda i, j: (i, 0)),
        pl.BlockSpec((BN, 1), lambda i, j: (i, 0)),
    ],
    out_shape=[
        jax.ShapeDtypeStruct((NT, KK), jnp.float32),
        jax.ShapeDtypeStruct((NT, 1), jnp.int32),
        jax.ShapeDtypeStruct((NT, 1), jnp.float32),
    ],
    scratch_shapes=[
        pltpu.VMEM((BN, 1), jnp.float32),
        pltpu.VMEM((BN, 1), jnp.int32),
    ],
)


_NW = 32
BPW = NT // _NW
NCH = BPW // 128


@functools.cache
def _build_sc_gather_hist():
    info = plsc.get_sparse_core_info()
    nc, ns = info.num_cores, info.num_subcores
    assert nc * ns == _NW
    mesh = plsc.VectorSubcoreMesh(core_axis_name="c", subcore_axis_name="s")

    kps = KK // ns

    @functools.partial(
        pl.kernel,
        mesh=mesh,
        out_type=[
            jax.ShapeDtypeStruct((NT, DD), jnp.float32),
            jax.ShapeDtypeStruct((nc, KK), jnp.float32),
        ],
        scratch_types=[
            pltpu.VMEM((NCH, 128), jnp.int32),
            pltpu.VMEM((BPW, DD), jnp.float32),
            pltpu.VMEM((128,), jnp.float32),
            pltpu.VMEM((kps,), jnp.float32),
            pltpu.VMEM_SHARED((KK,), jnp.float32),
            pltpu.SemaphoreType.DMA,
        ],
    )
    def _sc_gather_hist(emb_hbm, idx_hbm, out_hbm, cnt_hbm,
                        idx_v, rows_v, ones_v, zbuf_v, hist_sh, sem):
        c = lax.axis_index("c")
        s = lax.axis_index("s")
        wid = c * ns + s

        pltpu.sync_copy(idx_hbm.at[wid], idx_v)

        def _ones_body(t, carry):
            ones_v[pl.ds(t * 16, 16)] = jnp.full((16,), 1.0, jnp.float32)
            return carry
        lax.fori_loop(0, 128 // 16, _ones_body, 0)

        def _z_body(t, carry):
            zbuf_v[pl.ds(t * 16, 16)] = jnp.zeros((16,), jnp.float32)
            return carry
        lax.fori_loop(0, kps // 16, _z_body, 0)
        pltpu.sync_copy(zbuf_v, hist_sh.at[pl.ds(s * kps, kps)])

        copies = []
        for ch in range(NCH):
            copies.append(pltpu.async_copy(
                emb_hbm.at[idx_v.at[ch]], rows_v.at[pl.ds(ch * 128, 128)], sem))
        for cp in copies:
            cp.wait()
        pltpu.sync_copy(rows_v, out_hbm.at[pl.ds(wid * BPW, BPW)])

        plsc.subcore_barrier()
        for ch in range(NCH):
            pltpu.sync_copy(ones_v, hist_sh.at[idx_v.at[ch]], add=True)
        plsc.subcore_barrier()

        @pl.when(s == 0)
        def _():
            pltpu.sync_copy(hist_sh, cnt_hbm.at[c])

    return _sc_gather_hist


def _final_body(minv_ref, cnt_ref, loss_ref, perp_ref):
    cnt = cnt_ref[0:1, :] + cnt_ref[1:2, :]
    avg = cnt * (1.0 / NT)
    ent = jnp.sum(avg * jnp.log(avg + 1e-10))
    perp_ref[0, 0] = jnp.exp(-ent)
    m = jnp.sum(minv_ref[...]) / (NT * DD)
    loss_ref[0, 0] = BETA * m + DELTA * m


_final = pl.pallas_call(
    _final_body,
    in_specs=[
        pl.BlockSpec(memory_space=pltpu.VMEM),
        pl.BlockSpec(memory_space=pltpu.VMEM),
    ],
    out_specs=[
        pl.BlockSpec(memory_space=pltpu.SMEM),
        pl.BlockSpec(memory_space=pltpu.SMEM),
    ],
    out_shape=[
        jax.ShapeDtypeStruct((1, 1), jnp.float32),
        jax.ShapeDtypeStruct((1, 1), jnp.float32),
    ],
)


def kernel(latents_mean, embedding_weight):
    latents_shape = latents_mean.shape
    flat = latents_mean.reshape(NT, DD)
    zsq = jnp.sum(flat ** 2, axis=1, keepdims=True)
    esq = jnp.sum(embedding_weight ** 2, axis=1)[None, :]

    dist, inds, minv = _dist_argmin(zsq, esq, flat, embedding_weight)

    idx3 = inds.reshape(_NW, NCH, 128)
    quantized, counts = _build_sc_gather_hist()(embedding_weight, idx3)

    loss, perp = _final(minv, counts)

    return (quantized.reshape(latents_shape), loss.reshape(()),
            perp.reshape(()), inds, dist)
